# baseline (speedup 1.0000x reference)
import functools

import jax
import jax.numpy as jnp
import numpy as np
from jax import lax
from jax.experimental import pallas as pl
from jax.experimental.pallas import tpu as pltpu
from jax.experimental.pallas import tpu_sc as plsc

N_NODES = 10000
N_EDGES = 160000
D = 128
BN_EPS = 1e-5
LN_EPS = 1e-5
_BN_SCALE = np.float32(1.0 / np.sqrt(1.0 + BN_EPS))
_INV_SQRT2 = np.float32(1.0 / np.sqrt(2.0))

_NC = 2
_NS = 16
_NW = _NC * _NS
_CH = 128
_CPW = 40
_NCHUNK = _NW * _CPW
E_PAD = _NCHUNK * _CH
_CH2 = 64
_CPW2 = 80
_NCHUNK2 = _NW * _CPW2
_ACC = N_NODES + 16


def _gelu(x):
    return 0.5 * x * (1.0 + lax.erf(x * _INV_SQRT2))


def _bn(x, g, b):
    return (x * _BN_SCALE) * g + b


def _ln(x, g, b):
    mu = jnp.mean(x, axis=-1, keepdims=True)
    d = x - mu
    var = jnp.mean(d * d, axis=-1, keepdims=True)
    return d * lax.rsqrt(var + LN_EPS) * g + b


def _mm(a, b):
    return jax.lax.dot_general(a, b, (((1,), (0,)), ((), ())),
                               preferred_element_type=jnp.float32)


def _node_mlp(x, p, g1, be1, g2, be2, wpa, wpb):
    def body(x_ref, w1, b1, g1r, be1r, w2, b2, g2r, be2r, wpa_r, wpb_r,
             o_ref, ta_ref, tb_ref):
        h = _gelu(_bn(_mm(x_ref[...], w1[...]) + b1[...], g1r[...], be1r[...]))
        h = _gelu(_bn(_mm(h, w2[...]) + b2[...], g2r[...], be2r[...]))
        o_ref[...] = h
        ta_ref[...] = _mm(h, wpa_r[...])
        tb_ref[...] = _mm(h, wpb_r[...])

    out = jax.ShapeDtypeStruct((N_NODES, D), jnp.float32)
    return pl.pallas_call(
        body,
        out_shape=(out, out, out),
    )(x, p["l1"]["W"], p["l1"]["b"].reshape(1, D), g1, be1,
      p["l2"]["W"], p["l2"]["b"].reshape(1, D), g2, be2, wpa, wpb)


def _edge_mlp(edge_attr, pn, pcat, emb):
    E_T = 2048
    grid = (E_PAD // E_T,)

    def body(a_ref, w1, b1, g1, be1, w2, b2, g2, be2,
             embcat_ref, g0c, be0c, wc, bc, g1c, be1c, o_ref):
        a = a_ref[...]
        numer = a[:, : 32 - 4]
        h = _gelu(_bn(_mm(numer, w1[...]) + b1[...], g1[...], be1[...]))
        en = _gelu(_bn(_mm(h, w2[...]) + b2[...], g2[...], be2[...]))
        cat = a[:, 32 - 4:].astype(jnp.int32)
        ohs = []
        for i in range(4):
            ci = cat[:, i][:, None]
            oh = (ci == lax.broadcasted_iota(jnp.int32, (E_T, 8), 1))
            ohs.append(oh.astype(jnp.float32))
        oh_all = jnp.concatenate(ohs, axis=-1)
        ec = _mm(oh_all, embcat_ref[...])
        ec = _gelu(_bn(ec, g0c[...], be0c[...]))
        ec = _gelu(_bn(_mm(ec, wc[...]) + bc[...], g1c[...], be1c[...]))
        o_ref[...] = jnp.concatenate([en, ec], axis=-1)

    embcat = jnp.zeros((32, 32), jnp.float32)
    for i in range(4):
        embcat = lax.dynamic_update_slice(embcat, emb[i], (8 * i, 8 * i))

    espec = pl.BlockSpec((E_T, 32), lambda i: (i, 0))
    ospec = pl.BlockSpec((E_T, D), lambda i: (i, 0))
    full = lambda *s: pl.BlockSpec(s, lambda i: tuple(0 for _ in s))
    return pl.pallas_call(
        body,
        grid=grid,
        in_specs=[espec,
                  full(28, 96), full(1, 96), full(1, 96), full(1, 96),
                  full(96, 96), full(1, 96), full(1, 96), full(1, 96),
                  full(32, 32), full(1, 32), full(1, 32),
                  full(32, 32), full(1, 32), full(1, 32), full(1, 32)],
        out_specs=ospec,
        out_shape=jax.ShapeDtypeStruct((E_PAD, D), jnp.float32),
    )(edge_attr,
      pn["l1"]["W"], pn["l1"]["b"].reshape(1, 96), pn["n1"]["g"].reshape(1, 96), pn["n1"]["be"].reshape(1, 96),
      pn["l2"]["W"], pn["l2"]["b"].reshape(1, 96), pn["n2"]["g"].reshape(1, 96), pn["n2"]["be"].reshape(1, 96),
      embcat, pcat["n0"]["g"].reshape(1, 32), pcat["n0"]["be"].reshape(1, 32),
      pcat["l"]["W"], pcat["l"]["b"].reshape(1, 32),
      pcat["n1"]["g"].reshape(1, 32), pcat["n1"]["be"].reshape(1, 32))


def _edge_stage(edge_emb, s, scol, w, b, g, be, residual):
    E_T = 2048
    grid = (E_PAD // E_T,)

    def body(e_ref, s_ref, w_ref, b_ref, g_ref, be_ref, o_ref):
        e = e_ref[...]
        y = _gelu(_ln(_mm(e, w_ref[...]) + s_ref[...] + b_ref[...],
                      g_ref[...], be_ref[...]))
        o_ref[...] = e + y if residual else y

    espec = pl.BlockSpec((E_T, D), lambda i: (i, 0))
    sspec = pl.BlockSpec((E_T, D), lambda i: (i, scol))
    full = lambda *sh: pl.BlockSpec(sh, lambda i: tuple(0 for _ in sh))
    return pl.pallas_call(
        body,
        grid=grid,
        in_specs=[espec, sspec, full(D, D), full(1, D), full(1, D), full(1, D)],
        out_specs=espec,
        out_shape=jax.ShapeDtypeStruct((E_PAD, D), jnp.float32),
    )(edge_emb, s, w, b.reshape(1, D), g.reshape(1, D), be.reshape(1, D))


def _node_update(node_emb, aggp, wn1, wn2, b, g, be, wpa, wpb):
    def body(n_ref, a_ref, w1, w2, b_ref, g_ref, be_ref, wpa_r, wpb_r,
             o_ref, ta_ref, tb_ref):
        n = n_ref[...]
        agg = a_ref[0] + a_ref[1]
        u = _gelu(_ln(_mm(n, w1[...]) + _mm(agg, w2[...]) + b_ref[...],
                      g_ref[...], be_ref[...]))
        nn = n + u
        o_ref[...] = nn
        ta_ref[...] = _mm(nn, wpa_r[...])
        tb_ref[...] = _mm(nn, wpb_r[...])

    out = jax.ShapeDtypeStruct((N_NODES, D), jnp.float32)
    out2 = jax.ShapeDtypeStruct((N_NODES, 2 * D), jnp.float32)
    return pl.pallas_call(body, out_shape=(out, out2, out2))(
        node_emb, aggp, wn1, wn2, b.reshape(1, D), g.reshape(1, D),
        be.reshape(1, D), wpa, wpb)


def _node_update_f(node_emb, aggp, wn1, wn2, b, g, be, node_pre,
                   wpa1, wpa2, wpb1, wpb2):
    def body(n_ref, a_ref, w1, w2, b_ref, g_ref, be_ref, p_ref,
             wpa1_r, wpa2_r, wpb1_r, wpb2_r, ta_ref, tb_ref):
        n = n_ref[...]
        agg = a_ref[0] + a_ref[1]
        u = _gelu(_ln(_mm(n, w1[...]) + _mm(agg, w2[...]) + b_ref[...],
                      g_ref[...], be_ref[...]))
        nn = n + u
        nf = nn + p_ref[...]
        ta_ref[...] = jnp.concatenate(
            [_mm(nn, wpa1_r[...]), _mm(nf, wpa2_r[...])], axis=-1)
        tb_ref[...] = jnp.concatenate(
            [_mm(nn, wpb1_r[...]), _mm(nf, wpb2_r[...])], axis=-1)

    out2 = jax.ShapeDtypeStruct((N_NODES, 2 * D), jnp.float32)
    return pl.pallas_call(body, out_shape=(out2, out2))(
        node_emb, aggp, wn1, wn2, b.reshape(1, D), g.reshape(1, D),
        be.reshape(1, D), node_pre, wpa1, wpa2, wpb1, wpb2)


def _final_stage(edge_emb, edge_pre, s, scol, w_e, b1, g1, be1, w2, b2):
    E_T = 2048
    grid = (E_PAD // E_T,)

    def body(e_ref, ep_ref, s_ref, w_ref, b1_ref, g_ref, be_ref,
             w2_ref, b2_ref, o_ref):
        e = e_ref[...] + ep_ref[...]
        h = _gelu(_bn(_mm(e, w_ref[...]) + s_ref[...] + b1_ref[...],
                      g_ref[...], be_ref[...]))
        o_ref[...] = _mm(h, w2_ref[...]) + b2_ref[...]

    espec = pl.BlockSpec((E_T, D), lambda i: (i, 0))
    sspec = pl.BlockSpec((E_T, D), lambda i: (i, scol))
    full = lambda *sh: pl.BlockSpec(sh, lambda i: tuple(0 for _ in sh))
    return pl.pallas_call(
        body,
        grid=grid,
        in_specs=[espec, espec, sspec, full(D, D), full(1, D), full(1, D),
                  full(1, D), full(D, 3), full(1, 3)],
        out_specs=pl.BlockSpec((E_T, 3), lambda i: (i, 0)),
        out_shape=jax.ShapeDtypeStruct((E_PAD, 3), jnp.float32),
    )(edge_emb, edge_pre, s, w_e, b1.reshape(1, D), g1.reshape(1, D),
      be1.reshape(1, D), w2, b2.reshape(1, 3))


@functools.cache
def _sc_kernels():
    mesh = plsc.VectorSubcoreMesh(
        core_axis_name="c", subcore_axis_name="s",
        num_cores=_NC, num_subcores=_NS)


    @functools.partial(
        pl.kernel,
        out_type=jax.ShapeDtypeStruct((E_PAD, D), jnp.float32),
        mesh=mesh,
        scratch_types=[
            pltpu.VMEM((_CPW, _CH), jnp.int32),
            pltpu.VMEM((_CPW, _CH), jnp.int32),
            pltpu.VMEM((_CH, D), jnp.float32),
            pltpu.VMEM((_CH, D), jnp.float32),
            pltpu.VMEM((_CH, D), jnp.float32),
            pltpu.VMEM((_CH, D), jnp.float32),
            pltpu.VMEM((_CH, D), jnp.float32),
            pltpu.VMEM((_CH, D), jnp.float32),
        ] + [pltpu.SemaphoreType.DMA] * 9,
    )
    def gather2add(ta_hbm, tb_hbm, ia_hbm, ib_hbm, out_hbm,
                   ia_v, ib_v, ba0, bb0, ba1, bb1, ba2, bb2,
                   sa0, sb0, sa1, sb1, sa2, sb2, so0, so1, so2):
        c = lax.axis_index("c")
        s = lax.axis_index("s")
        w = s * _NC + c
        first = w * _CPW
        pltpu.sync_copy(ia_hbm.at[pl.ds(first, _CPW)], ia_v)
        pltpu.sync_copy(ib_hbm.at[pl.ds(first, _CPW)], ib_v)

        ba = [ba0, ba1, ba2]
        bb = [bb0, bb1, bb2]
        sa = [sa0, sa1, sa2]
        sb = [sb0, sb1, sb2]
        so = [so0, so1, so2]

        def start(k):
            t = k % 3
            pltpu.async_copy(ta_hbm.at[ia_v.at[k]], ba[t], sa[t])
            pltpu.async_copy(tb_hbm.at[ib_v.at[k]], bb[t], sb[t])

        def finish(k):
            t = k % 3
            pltpu.make_async_copy(ta_hbm.at[ia_v.at[k]], ba[t], sa[t]).wait()
            pltpu.make_async_copy(tb_hbm.at[ib_v.at[k]], bb[t], sb[t]).wait()
            bav, bbv = ba[t], bb[t]

            def addrow(r, carry):
                for cc in range(D // 16):
                    sl = pl.ds(cc * 16, 16)
                    plsc.addupdate(bav.at[r, sl], bbv[r, sl])
                return carry

            lax.fori_loop(0, _CH, addrow, 0)
            pltpu.async_copy(bav, out_hbm.at[pl.ds((first + k) * _CH, _CH)],
                             so[t])

        def drain_out(k):
            t = k % 3
            pltpu.make_async_copy(ba[t], out_hbm.at[pl.ds(0, _CH)],
                                  so[t]).wait()

        for k in range(_CPW + 2):
            if k < _CPW:
                if k >= 3:
                    drain_out(k - 3)
                start(k)
            if k >= 2:
                finish(k - 2)
        for k in range(_CPW - 3, _CPW):
            drain_out(k)

    @functools.partial(
        pl.kernel,
        out_type=jax.ShapeDtypeStruct((E_PAD, 2 * D), jnp.float32),
        mesh=mesh,
        scratch_types=[
            pltpu.VMEM((_CPW2, _CH2), jnp.int32),
            pltpu.VMEM((_CPW2, _CH2), jnp.int32),
            pltpu.VMEM((_CH2, 2 * D), jnp.float32),
            pltpu.VMEM((_CH2, 2 * D), jnp.float32),
            pltpu.VMEM((_CH2, 2 * D), jnp.float32),
            pltpu.VMEM((_CH2, 2 * D), jnp.float32),
            pltpu.VMEM((_CH2, 2 * D), jnp.float32),
            pltpu.VMEM((_CH2, 2 * D), jnp.float32),
        ] + [pltpu.SemaphoreType.DMA] * 9,
    )
    def gather2add256(ta_hbm, tb_hbm, ia_hbm, ib_hbm, out_hbm,
                      ia_v, ib_v, ba0, bb0, ba1, bb1, ba2, bb2,
                      sa0, sb0, sa1, sb1, sa2, sb2, so0, so1, so2):
        c = lax.axis_index("c")
        s = lax.axis_index("s")
        w = s * _NC + c
        first = w * _CPW2
        pltpu.sync_copy(ia_hbm.at[pl.ds(first, _CPW2)], ia_v)
        pltpu.sync_copy(ib_hbm.at[pl.ds(first, _CPW2)], ib_v)

        ba = [ba0, ba1, ba2]
        bb = [bb0, bb1, bb2]
        sa = [sa0, sa1, sa2]
        sb = [sb0, sb1, sb2]
        so = [so0, so1, so2]

        def start(t, k):
            pltpu.async_copy(ta_hbm.at[ia_v.at[k]], ba[t], sa[t])
            pltpu.async_copy(tb_hbm.at[ib_v.at[k]], bb[t], sb[t])

        def finish(t, k):
            pltpu.make_async_copy(ta_hbm.at[ia_v.at[k]], ba[t], sa[t]).wait()
            pltpu.make_async_copy(tb_hbm.at[ib_v.at[k]], bb[t], sb[t]).wait()
            bav, bbv = ba[t], bb[t]

            def addrow(r, carry):
                for cc in range(2 * D // 16):
                    sl = pl.ds(cc * 16, 16)
                    plsc.addupdate(bav.at[r, sl], bbv[r, sl])
                return carry

            lax.fori_loop(0, _CH2, addrow, 0)
            pltpu.async_copy(bav, out_hbm.at[pl.ds((first + k) * _CH2, _CH2)],
                             so[t])

        def wait_out(t):
            pltpu.make_async_copy(ba[t], out_hbm.at[pl.ds(0, _CH2)],
                                  so[t]).wait()

        start(0, 0)
        start(1, 1)
        start(2, 2)

        def rnd(r, carry):
            k = 3 * r
            for t in range(3):
                finish(t, k + t)
            for t in range(3):
                wait_out(t)
                start(t, k + 3 + t)
            return carry

        lax.fori_loop(0, (_CPW2 - 5) // 3, rnd, 0)
        finish(0, _CPW2 - 5)
        wait_out(0)
        start(0, _CPW2 - 2)
        finish(1, _CPW2 - 4)
        wait_out(1)
        start(1, _CPW2 - 1)
        finish(2, _CPW2 - 3)
        finish(0, _CPW2 - 2)
        finish(1, _CPW2 - 1)
        wait_out(2)
        wait_out(0)
        wait_out(1)

    @functools.partial(
        pl.kernel,
        out_type=jax.ShapeDtypeStruct((_NC, N_NODES, D), jnp.float32),
        mesh=mesh,
        scratch_types=[
            pltpu.VMEM((_CPW, _CH), jnp.int32),
            pltpu.VMEM((_CH, D), jnp.float32),
            pltpu.VMEM((_CH, D), jnp.float32),
            pltpu.VMEM_SHARED((_ACC, D), jnp.float32),
        ] + [pltpu.SemaphoreType.DMA] * 4,
    )
    def segsum(m_hbm, dst_hbm, out_hbm, idx_v, bm0, bm1, acc_sh,
               sm0, sm1, ss0, ss1):
        c = lax.axis_index("c")
        s = lax.axis_index("s")
        w = s * _NC + c
        first = w * _CPW

        def zrow(r, carry):
            for cc in range(D // 16):
                bm0[r, pl.ds(cc * 16, 16)] = jnp.zeros((16,), jnp.float32)
            return carry

        lax.fori_loop(0, _CH, zrow, 0)
        row0 = s * 624
        for k in range(4):
            pltpu.sync_copy(bm0, acc_sh.at[pl.ds(row0 + k * _CH, _CH)])
        pltpu.sync_copy(bm0.at[pl.ds(0, 112)],
                        acc_sh.at[pl.ds(row0 + 512, 112)])

        @pl.when(s == _NS - 1)
        def _zero_tail():
            pltpu.sync_copy(bm0.at[pl.ds(0, 32)],
                            acc_sh.at[pl.ds(_ACC - 32, 32)])

        pltpu.sync_copy(dst_hbm.at[pl.ds(first, _CPW)], idx_v)
        plsc.subcore_barrier()

        bm = [bm0, bm1]
        sm = [sm0, sm1]
        ss = [ss0, ss1]

        def load(k):
            t = k % 2
            pltpu.async_copy(m_hbm.at[pl.ds((first + k) * _CH, _CH)],
                             bm[t], sm[t])

        def scat(k):
            t = k % 2
            pltpu.make_async_copy(m_hbm.at[pl.ds(0, _CH)], bm[t], sm[t]).wait()
            pltpu.async_copy(bm[t], acc_sh.at[idx_v.at[k]], ss[t], add=True)

        def drain_scat(k):
            t = k % 2
            pltpu.make_async_copy(bm[t], acc_sh.at[pl.ds(0, _CH)],
                                  ss[t]).wait()

        for k in range(_CPW + 1):
            if k < _CPW:
                if k >= 2:
                    drain_scat(k - 2)
                load(k)
            if k >= 1:
                scat(k - 1)
        for k in range(_CPW - 2, _CPW):
            drain_scat(k)
        plsc.subcore_barrier()

        def out_rows(r, n):
            pltpu.sync_copy(acc_sh.at[pl.ds(r, n)], bm0.at[pl.ds(0, n)])
            pltpu.sync_copy(bm0.at[pl.ds(0, n)], out_hbm.at[c, pl.ds(r, n)])

        for k in range(4):
            out_rows(row0 + k * _CH, _CH)
        out_rows(row0 + 512, 112)

        @pl.when(s == _NS - 1)
        def _out_tail():
            out_rows(N_NODES - 16, 16)

    return gather2add, gather2add256, segsum


def _sc_gather2add(ta, tb, ia, ib):
    return _sc_kernels()[0](ta, tb, ia, ib)


def _sc_gather2add256(ta, tb, ia, ib):
    return _sc_kernels()[1](ta, tb, ia, ib)


def _sc_segsum(m, dst):
    return _sc_kernels()[2](m, dst)


def kernel(x, edge_attr, params, edge_index):
    pad_n = E_PAD - N_EDGES
    spread = (jnp.arange(pad_n, dtype=jnp.int32) * 37) % N_NODES
    src_f = jnp.concatenate([edge_index[0], spread])
    dst_f = jnp.concatenate([edge_index[1], spread])
    src = src_f.reshape(_NCHUNK, _CH)
    dst = dst_f.reshape(_NCHUNK, _CH)
    src64 = src_f.reshape(_NCHUNK2, _CH2)
    dst64 = dst_f.reshape(_NCHUNK2, _CH2)
    dst_s = jnp.concatenate(
        [edge_index[1],
         N_NODES + (jnp.arange(pad_n, dtype=jnp.int32) % 16)]
    ).reshape(_NCHUNK, _CH)
    edge_attr = jnp.pad(edge_attr, ((0, pad_n), (0, 0)))

    gnn = params["gnn"]
    msgW = [lp["msg"]["W"] for lp in gnn]
    edgW = [lp["edge"]["W"] for lp in gnn]
    nodW = [lp["node"]["W"] for lp in gnn]
    finW = params["final"]["l1"]["W"]

    pm = params["node_mlp"]
    node_emb, ta, tb = _node_mlp(
        x, pm, pm["n1"]["g"].reshape(1, D), pm["n1"]["be"].reshape(1, D),
        pm["n2"]["g"].reshape(1, D), pm["n2"]["be"].reshape(1, D),
        msgW[0][0:D], msgW[0][D:2 * D])
    node_pre = node_emb

    edge_emb = _edge_mlp(edge_attr, params["edge_numer"],
                         params["edge_cat_mlp"], params["cat_emb"])
    edge_pre = edge_emb

    s1 = _sc_gather2add(ta, tb, dst, src)
    s1col = 0
    for l in range(3):
        lp = gnn[l]
        m = _edge_stage(edge_emb, s1, s1col, msgW[l][2 * D:3 * D],
                        lp["msg"]["b"], lp["msg"]["g"], lp["msg"]["be"],
                        residual=False)
        aggp = _sc_segsum(m, dst_s)
        if l < 2:
            wpa = jnp.concatenate([edgW[l][D:2 * D], msgW[l + 1][0:D]], axis=1)
            wpb = jnp.concatenate([edgW[l][2 * D:3 * D], msgW[l + 1][D:2 * D]],
                                  axis=1)
            node_emb, Ta, Tb = _node_update(
                node_emb, aggp, nodW[l][0:D], nodW[l][D:2 * D],
                lp["node"]["b"], lp["node"]["g"], lp["node"]["be"], wpa, wpb)
        else:
            Ta, Tb = _node_update_f(
                node_emb, aggp, nodW[l][0:D], nodW[l][D:2 * D],
                lp["node"]["b"], lp["node"]["g"], lp["node"]["be"], node_pre,
                edgW[l][D:2 * D], finW[D:2 * D],
                edgW[l][2 * D:3 * D], finW[0:D])
        g2 = _sc_gather2add256(Ta, Tb, dst64, src64)
        edge_emb = _edge_stage(edge_emb, g2, 0, edgW[l][0:D], lp["edge"]["b"],
                               lp["edge"]["g"], lp["edge"]["be"],
                               residual=True)
        s1, s1col = g2, 1

    pf = params["final"]
    out = _final_stage(edge_emb, edge_pre, s1, 1, finW[2 * D:3 * D],
                       pf["l1"]["b"], pf["n1"]["g"], pf["n1"]["be"],
                       pf["l2"]["W"], pf["l2"]["b"])
    return out[:N_EDGES]


# speedup vs baseline: 1.1179x; 1.1179x over previous
import functools

import jax
import jax.numpy as jnp
import numpy as np
from jax import lax
from jax.experimental import pallas as pl
from jax.experimental.pallas import tpu as pltpu
from jax.experimental.pallas import tpu_sc as plsc

N_NODES = 10000
N_EDGES = 160000
D = 128
BN_EPS = 1e-5
LN_EPS = 1e-5
_BN_SCALE = np.float32(1.0 / np.sqrt(1.0 + BN_EPS))
_INV_SQRT2 = np.float32(1.0 / np.sqrt(2.0))

_NC = 2
_NS = 16
_NW = _NC * _NS
E_PAD = 163840
E_H = E_PAD // 2
_CPW_SS = 20
_NCH_SS = _NW * _CPW_SS
_CPW_G1 = 20
_NCH_G1 = _NW * _CPW_G1
_CPW_G2 = 40
_NCH_G2 = _NW * _CPW_G2
_ACC = N_NODES + 16


def _gelu(x):
    return 0.5 * x * (1.0 + lax.erf(x * _INV_SQRT2))


def _bn(x, g, b):
    return (x * _BN_SCALE) * g + b


def _ln(x, g, b):
    mu = jnp.mean(x, axis=-1, keepdims=True)
    d = x - mu
    var = jnp.mean(d * d, axis=-1, keepdims=True)
    return d * lax.rsqrt(var + LN_EPS) * g + b


def _mm(a, b):
    return jax.lax.dot_general(a, b, (((1,), (0,)), ((), ())),
                               preferred_element_type=jnp.float32)


def _node_mlp(x, p, g1, be1, g2, be2, wpa, wpb):
    def body(x_ref, w1, b1, g1r, be1r, w2, b2, g2r, be2r, wpa_r, wpb_r,
             o_ref, ta_ref, tb_ref):
        h = _gelu(_bn(_mm(x_ref[...], w1[...]) + b1[...], g1r[...], be1r[...]))
        h = _gelu(_bn(_mm(h, w2[...]) + b2[...], g2r[...], be2r[...]))
        o_ref[...] = h
        ta_ref[...] = _mm(h, wpa_r[...])
        tb_ref[...] = _mm(h, wpb_r[...])

    out = jax.ShapeDtypeStruct((N_NODES, D), jnp.float32)
    return pl.pallas_call(
        body,
        out_shape=(out, out, out),
    )(x, p["l1"]["W"], p["l1"]["b"].reshape(1, D), g1, be1,
      p["l2"]["W"], p["l2"]["b"].reshape(1, D), g2, be2, wpa, wpb)


def _edge_mlp(edge_attr, pn, pcat, emb):
    E = edge_attr.shape[0]
    E_T = 2048
    grid = (E // E_T,)

    def body(a_ref, w1, b1, g1, be1, w2, b2, g2, be2,
             embcat_ref, g0c, be0c, wc, bc, g1c, be1c, o_ref):
        a = a_ref[...]
        numer = a[:, : 32 - 4]
        h = _gelu(_bn(_mm(numer, w1[...]) + b1[...], g1[...], be1[...]))
        en = _gelu(_bn(_mm(h, w2[...]) + b2[...], g2[...], be2[...]))
        cat = a[:, 32 - 4:].astype(jnp.int32)
        ohs = []
        for i in range(4):
            ci = cat[:, i][:, None]
            oh = (ci == lax.broadcasted_iota(jnp.int32, (E_T, 8), 1))
            ohs.append(oh.astype(jnp.float32))
        oh_all = jnp.concatenate(ohs, axis=-1)
        ec = _mm(oh_all, embcat_ref[...])
        ec = _gelu(_bn(ec, g0c[...], be0c[...]))
        ec = _gelu(_bn(_mm(ec, wc[...]) + bc[...], g1c[...], be1c[...]))
        o_ref[...] = jnp.concatenate([en, ec], axis=-1)

    embcat = jnp.zeros((32, 32), jnp.float32)
    for i in range(4):
        embcat = lax.dynamic_update_slice(embcat, emb[i], (8 * i, 8 * i))

    espec = pl.BlockSpec((E_T, 32), lambda i: (i, 0))
    ospec = pl.BlockSpec((E_T, D), lambda i: (i, 0))
    full = lambda *s: pl.BlockSpec(s, lambda i: tuple(0 for _ in s))
    return pl.pallas_call(
        body,
        grid=grid,
        in_specs=[espec,
                  full(28, 96), full(1, 96), full(1, 96), full(1, 96),
                  full(96, 96), full(1, 96), full(1, 96), full(1, 96),
                  full(32, 32), full(1, 32), full(1, 32),
                  full(32, 32), full(1, 32), full(1, 32), full(1, 32)],
        out_specs=ospec,
        out_shape=jax.ShapeDtypeStruct((E, D), jnp.float32),
    )(edge_attr,
      pn["l1"]["W"], pn["l1"]["b"].reshape(1, 96), pn["n1"]["g"].reshape(1, 96), pn["n1"]["be"].reshape(1, 96),
      pn["l2"]["W"], pn["l2"]["b"].reshape(1, 96), pn["n2"]["g"].reshape(1, 96), pn["n2"]["be"].reshape(1, 96),
      embcat, pcat["n0"]["g"].reshape(1, 32), pcat["n0"]["be"].reshape(1, 32),
      pcat["l"]["W"], pcat["l"]["b"].reshape(1, 32),
      pcat["n1"]["g"].reshape(1, 32), pcat["n1"]["be"].reshape(1, 32))


def _edge_stage(edge_emb, s, scol, w, b, g, be, residual):
    E = edge_emb.shape[0]
    E_T = 2048
    grid = (E // E_T,)

    def body(e_ref, s_ref, w_ref, b_ref, g_ref, be_ref, o_ref):
        e = e_ref[...]
        y = _gelu(_ln(_mm(e, w_ref[...]) + s_ref[...] + b_ref[...],
                      g_ref[...], be_ref[...]))
        o_ref[...] = e + y if residual else y

    espec = pl.BlockSpec((E_T, D), lambda i: (i, 0))
    sspec = pl.BlockSpec((E_T, D), lambda i: (i, scol))
    full = lambda *sh: pl.BlockSpec(sh, lambda i: tuple(0 for _ in sh))
    return pl.pallas_call(
        body,
        grid=grid,
        in_specs=[espec, sspec, full(D, D), full(1, D), full(1, D), full(1, D)],
        out_specs=espec,
        out_shape=jax.ShapeDtypeStruct((E, D), jnp.float32),
    )(edge_emb, s, w, b.reshape(1, D), g.reshape(1, D), be.reshape(1, D))


def _node_update(node_emb, aggA, aggB, wn1, wn2, b, g, be, wpa, wpb):
    N_T = 2000
    grid = (N_NODES // N_T,)

    def body(n_ref, a_ref, b2_ref, w1, w2, b_ref, g_ref, be_ref,
             wpa_r, wpb_r, o_ref, ta_ref, tb_ref):
        n = n_ref[...]
        agg = a_ref[0] + a_ref[1] + b2_ref[0] + b2_ref[1]
        u = _gelu(_ln(_mm(n, w1[...]) + _mm(agg, w2[...]) + b_ref[...],
                      g_ref[...], be_ref[...]))
        nn = n + u
        o_ref[...] = nn
        ta_ref[...] = _mm(nn, wpa_r[...])
        tb_ref[...] = _mm(nn, wpb_r[...])

    nspec = pl.BlockSpec((N_T, D), lambda i: (i, 0))
    aspec = pl.BlockSpec((2, N_T, D), lambda i: (0, i, 0))
    t2spec = pl.BlockSpec((N_T, 2 * D), lambda i: (i, 0))
    full = lambda *sh: pl.BlockSpec(sh, lambda i: tuple(0 for _ in sh))
    return pl.pallas_call(
        body,
        grid=grid,
        in_specs=[nspec, aspec, aspec, full(D, D), full(D, D), full(1, D),
                  full(1, D), full(1, D), full(D, 2 * D), full(D, 2 * D)],
        out_specs=(nspec, t2spec, t2spec),
        out_shape=(jax.ShapeDtypeStruct((N_NODES, D), jnp.float32),
                   jax.ShapeDtypeStruct((N_NODES, 2 * D), jnp.float32),
                   jax.ShapeDtypeStruct((N_NODES, 2 * D), jnp.float32)),
    )(node_emb, aggA, aggB, wn1, wn2, b.reshape(1, D), g.reshape(1, D),
      be.reshape(1, D), wpa, wpb)


def _node_update_f(node_emb, aggA, aggB, wn1, wn2, b, g, be, node_pre,
                   wpa1, wpa2, wpb1, wpb2):
    N_T = 2000
    grid = (N_NODES // N_T,)

    def body(n_ref, a_ref, b2_ref, w1, w2, b_ref, g_ref, be_ref, p_ref,
             wpa1_r, wpa2_r, wpb1_r, wpb2_r, ta_ref, tb_ref):
        n = n_ref[...]
        agg = a_ref[0] + a_ref[1] + b2_ref[0] + b2_ref[1]
        u = _gelu(_ln(_mm(n, w1[...]) + _mm(agg, w2[...]) + b_ref[...],
                      g_ref[...], be_ref[...]))
        nn = n + u
        nf = nn + p_ref[...]
        ta_ref[...] = jnp.concatenate(
            [_mm(nn, wpa1_r[...]), _mm(nf, wpa2_r[...])], axis=-1)
        tb_ref[...] = jnp.concatenate(
            [_mm(nn, wpb1_r[...]), _mm(nf, wpb2_r[...])], axis=-1)

    nspec = pl.BlockSpec((N_T, D), lambda i: (i, 0))
    aspec = pl.BlockSpec((2, N_T, D), lambda i: (0, i, 0))
    t2spec = pl.BlockSpec((N_T, 2 * D), lambda i: (i, 0))
    full = lambda *sh: pl.BlockSpec(sh, lambda i: tuple(0 for _ in sh))
    return pl.pallas_call(
        body,
        grid=grid,
        in_specs=[nspec, aspec, aspec, full(D, D), full(D, D), full(1, D),
                  full(1, D), full(1, D), nspec,
                  full(D, D), full(D, D), full(D, D), full(D, D)],
        out_specs=(t2spec, t2spec),
        out_shape=(jax.ShapeDtypeStruct((N_NODES, 2 * D), jnp.float32),
                   jax.ShapeDtypeStruct((N_NODES, 2 * D), jnp.float32)),
    )(node_emb, aggA, aggB, wn1, wn2, b.reshape(1, D), g.reshape(1, D),
      be.reshape(1, D), node_pre, wpa1, wpa2, wpb1, wpb2)


def _final_stage(edge_emb, edge_pre, s, scol, w_e, b1, g1, be1, w2, b2):
    E = edge_emb.shape[0]
    E_T = 2048
    grid = (E // E_T,)

    def body(e_ref, ep_ref, s_ref, w_ref, b1_ref, g_ref, be_ref,
             w2_ref, b2_ref, o_ref):
        e = e_ref[...] + ep_ref[...]
        h = _gelu(_bn(_mm(e, w_ref[...]) + s_ref[...] + b1_ref[...],
                      g_ref[...], be_ref[...]))
        o_ref[...] = _mm(h, w2_ref[...]) + b2_ref[...]

    espec = pl.BlockSpec((E_T, D), lambda i: (i, 0))
    sspec = pl.BlockSpec((E_T, D), lambda i: (i, scol))
    full = lambda *sh: pl.BlockSpec(sh, lambda i: tuple(0 for _ in sh))
    return pl.pallas_call(
        body,
        grid=grid,
        in_specs=[espec, espec, sspec, full(D, D), full(1, D), full(1, D),
                  full(1, D), full(D, 3), full(1, 3)],
        out_specs=pl.BlockSpec((E_T, 3), lambda i: (i, 0)),
        out_shape=jax.ShapeDtypeStruct((E, 3), jnp.float32),
    )(edge_emb, edge_pre, s, w_e, b1.reshape(1, D), g1.reshape(1, D),
      be1.reshape(1, D), w2, b2.reshape(1, 3))


@functools.cache
def _sc_kernels():
    mesh = plsc.VectorSubcoreMesh(
        core_axis_name="c", subcore_axis_name="s",
        num_cores=_NC, num_subcores=_NS)

    def make_gather(cpw, ch, width):
        @functools.partial(
            pl.kernel,
            out_type=jax.ShapeDtypeStruct((_NW * cpw * ch, width),
                                          jnp.float32),
            mesh=mesh,
            scratch_types=[
                pltpu.VMEM((cpw, ch), jnp.int32),
                pltpu.VMEM((cpw, ch), jnp.int32),
                pltpu.VMEM((ch, width), jnp.float32),
                pltpu.VMEM((ch, width), jnp.float32),
                pltpu.VMEM((ch, width), jnp.float32),
                pltpu.VMEM((ch, width), jnp.float32),
                pltpu.VMEM((ch, width), jnp.float32),
                pltpu.VMEM((ch, width), jnp.float32),
            ] + [pltpu.SemaphoreType.DMA] * 9,
        )
        def gather2add(ta_hbm, tb_hbm, ia_hbm, ib_hbm, out_hbm,
                       ia_v, ib_v, ba0, bb0, ba1, bb1, ba2, bb2,
                       sa0, sb0, sa1, sb1, sa2, sb2, so0, so1, so2):
            c = lax.axis_index("c")
            s = lax.axis_index("s")
            w = s * _NC + c
            first = w * cpw
            pltpu.sync_copy(ia_hbm.at[w], ia_v)
            pltpu.sync_copy(ib_hbm.at[w], ib_v)

            ba = [ba0, ba1, ba2]
            bb = [bb0, bb1, bb2]
            sa = [sa0, sa1, sa2]
            sb = [sb0, sb1, sb2]
            so = [so0, so1, so2]

            def start(k):
                t = k % 3
                pltpu.async_copy(ta_hbm.at[ia_v.at[k]], ba[t], sa[t])
                pltpu.async_copy(tb_hbm.at[ib_v.at[k]], bb[t], sb[t])

            def finish(k):
                t = k % 3
                pltpu.make_async_copy(ta_hbm.at[ia_v.at[k]], ba[t],
                                      sa[t]).wait()
                pltpu.make_async_copy(tb_hbm.at[ib_v.at[k]], bb[t],
                                      sb[t]).wait()
                bav, bbv = ba[t], bb[t]

                def addrow(r, carry):
                    for cc in range(width // 16):
                        sl = pl.ds(cc * 16, 16)
                        plsc.addupdate(bav.at[r, sl], bbv[r, sl])
                    return carry

                lax.fori_loop(0, ch, addrow, 0)
                pltpu.async_copy(bav, out_hbm.at[pl.ds((first + k) * ch, ch)],
                                 so[t])

            def drain_out(k):
                t = k % 3
                pltpu.make_async_copy(ba[t], out_hbm.at[pl.ds(0, ch)],
                                      so[t]).wait()

            for k in range(cpw + 2):
                if k < cpw:
                    if k >= 3:
                        drain_out(k - 3)
                    start(k)
                if k >= 2:
                    finish(k - 2)
            for k in range(cpw - 3, cpw):
                drain_out(k)

        return gather2add

    @functools.partial(
        pl.kernel,
        out_type=jax.ShapeDtypeStruct((_NC, N_NODES, D), jnp.float32),
        mesh=mesh,
        scratch_types=[
            pltpu.VMEM((_CPW_SS, 128), jnp.int32),
            pltpu.VMEM((128, D), jnp.float32),
            pltpu.VMEM((128, D), jnp.float32),
            pltpu.VMEM_SHARED((_ACC, D), jnp.float32),
        ] + [pltpu.SemaphoreType.DMA] * 4,
    )
    def segsum(m_hbm, dst_hbm, out_hbm, idx_v, bm0, bm1, acc_sh,
               sm0, sm1, ss0, ss1):
        c = lax.axis_index("c")
        s = lax.axis_index("s")
        w = s * _NC + c
        first = w * _CPW_SS

        def zrow(r, carry):
            for cc in range(D // 16):
                bm0[r, pl.ds(cc * 16, 16)] = jnp.zeros((16,), jnp.float32)
            return carry

        lax.fori_loop(0, 128, zrow, 0)
        row0 = s * 624
        for k in range(4):
            pltpu.sync_copy(bm0, acc_sh.at[pl.ds(row0 + k * 128, 128)])
        pltpu.sync_copy(bm0.at[pl.ds(0, 112)],
                        acc_sh.at[pl.ds(row0 + 512, 112)])

        @pl.when(s == _NS - 1)
        def _zero_tail():
            pltpu.sync_copy(bm0.at[pl.ds(0, 32)],
                            acc_sh.at[pl.ds(_ACC - 32, 32)])

        pltpu.sync_copy(dst_hbm.at[w], idx_v)
        plsc.subcore_barrier()

        bm = [bm0, bm1]
        sm = [sm0, sm1]
        ss = [ss0, ss1]

        def load(k):
            t = k % 2
            pltpu.async_copy(m_hbm.at[pl.ds((first + k) * 128, 128)],
                             bm[t], sm[t])

        def scat(k):
            t = k % 2
            pltpu.make_async_copy(m_hbm.at[pl.ds(0, 128)], bm[t], sm[t]).wait()
            pltpu.async_copy(bm[t], acc_sh.at[idx_v.at[k]], ss[t], add=True)

        def drain_scat(k):
            t = k % 2
            pltpu.make_async_copy(bm[t], acc_sh.at[pl.ds(0, 128)],
                                  ss[t]).wait()

        for k in range(_CPW_SS + 1):
            if k < _CPW_SS:
                if k >= 2:
                    drain_scat(k - 2)
                load(k)
            if k >= 1:
                scat(k - 1)
        for k in range(_CPW_SS - 2, _CPW_SS):
            drain_scat(k)
        plsc.subcore_barrier()

        def out_rows(r, n):
            pltpu.sync_copy(acc_sh.at[pl.ds(r, n)], bm0.at[pl.ds(0, n)])
            pltpu.sync_copy(bm0.at[pl.ds(0, n)], out_hbm.at[c, pl.ds(r, n)])

        for k in range(4):
            out_rows(row0 + k * 128, 128)
        out_rows(row0 + 512, 112)

        @pl.when(s == _NS - 1)
        def _out_tail():
            out_rows(N_NODES - 16, 16)

    return make_gather(_CPW_G1, 128, D), make_gather(_CPW_G2, 64, 2 * D), segsum


def _sc_gather2add(ta, tb, ia, ib):
    return _sc_kernels()[0](ta, tb, ia, ib)


def _sc_gather2add256(ta, tb, ia, ib):
    return _sc_kernels()[1](ta, tb, ia, ib)


def _sc_segsum(m, dst):
    return _sc_kernels()[2](m, dst)


def kernel(x, edge_attr, params, edge_index):
    pad_n = E_PAD - N_EDGES
    spread = (jnp.arange(pad_n, dtype=jnp.int32) * 37) % N_NODES
    src_f = jnp.concatenate([edge_index[0], spread])
    dst_f = jnp.concatenate([edge_index[1], spread])
    dsts_f = jnp.concatenate(
        [edge_index[1],
         N_NODES + (jnp.arange(pad_n, dtype=jnp.int32) % 16)])
    halves = []
    for h in range(2):
        sl = slice(h * E_H, (h + 1) * E_H)
        halves.append(dict(
            src=src_f[sl].reshape(_NW, _CPW_G1, 128),
            dst=dst_f[sl].reshape(_NW, _CPW_G1, 128),
            src64=src_f[sl].reshape(_NW, _CPW_G2, 64),
            dst64=dst_f[sl].reshape(_NW, _CPW_G2, 64),
            dsts=dsts_f[sl].reshape(_NW, _CPW_SS, 128),
        ))
    edge_attr = jnp.pad(edge_attr, ((0, pad_n), (0, 0)))

    gnn = params["gnn"]
    msgW = [lp["msg"]["W"] for lp in gnn]
    edgW = [lp["edge"]["W"] for lp in gnn]
    nodW = [lp["node"]["W"] for lp in gnn]
    finW = params["final"]["l1"]["W"]

    pm = params["node_mlp"]
    node_emb, ta, tb = _node_mlp(
        x, pm, pm["n1"]["g"].reshape(1, D), pm["n1"]["be"].reshape(1, D),
        pm["n2"]["g"].reshape(1, D), pm["n2"]["be"].reshape(1, D),
        msgW[0][0:D], msgW[0][D:2 * D])
    node_pre = node_emb

    ee = [_edge_mlp(edge_attr[h * E_H:(h + 1) * E_H], params["edge_numer"],
                    params["edge_cat_mlp"], params["cat_emb"])
          for h in range(2)]
    edge_pre = list(ee)

    s1 = [_sc_gather2add(ta, tb, halves[h]["dst"], halves[h]["src"])
          for h in range(2)]
    s1col = 0
    for l in range(3):
        lp = gnn[l]
        m0 = _edge_stage(ee[0], s1[0], s1col, msgW[l][2 * D:3 * D],
                         lp["msg"]["b"], lp["msg"]["g"], lp["msg"]["be"],
                         residual=False)
        aggA = _sc_segsum(m0, halves[0]["dsts"])
        m1 = _edge_stage(ee[1], s1[1], s1col, msgW[l][2 * D:3 * D],
                         lp["msg"]["b"], lp["msg"]["g"], lp["msg"]["be"],
                         residual=False)
        aggB = _sc_segsum(m1, halves[1]["dsts"])
        if l < 2:
            wpa = jnp.concatenate([edgW[l][D:2 * D], msgW[l + 1][0:D]], axis=1)
            wpb = jnp.concatenate([edgW[l][2 * D:3 * D], msgW[l + 1][D:2 * D]],
                                  axis=1)
            node_emb, Ta, Tb = _node_update(
                node_emb, aggA, aggB, nodW[l][0:D], nodW[l][D:2 * D],
                lp["node"]["b"], lp["node"]["g"], lp["node"]["be"], wpa, wpb)
        else:
            Ta, Tb = _node_update_f(
                node_emb, aggA, aggB, nodW[l][0:D], nodW[l][D:2 * D],
                lp["node"]["b"], lp["node"]["g"], lp["node"]["be"], node_pre,
                edgW[l][D:2 * D], finW[D:2 * D],
                edgW[l][2 * D:3 * D], finW[0:D])
        g0 = _sc_gather2add256(Ta, Tb, halves[0]["dst64"], halves[0]["src64"])
        ee[0] = _edge_stage(ee[0], g0, 0, edgW[l][0:D], lp["edge"]["b"],
                            lp["edge"]["g"], lp["edge"]["be"], residual=True)
        g1 = _sc_gather2add256(Ta, Tb, halves[1]["dst64"], halves[1]["src64"])
        ee[1] = _edge_stage(ee[1], g1, 0, edgW[l][0:D], lp["edge"]["b"],
                            lp["edge"]["g"], lp["edge"]["be"], residual=True)
        s1, s1col = [g0, g1], 1

    pf = params["final"]
    out = [_final_stage(ee[h], edge_pre[h], s1[h], 1, finW[2 * D:3 * D],
                        pf["l1"]["b"], pf["n1"]["g"], pf["n1"]["be"],
                        pf["l2"]["W"], pf["l2"]["b"])
           for h in range(2)]
    return jnp.concatenate(out)[:N_EDGES]


# speedup vs baseline: 1.1864x; 1.0613x over previous
import functools

import jax
import jax.numpy as jnp
import numpy as np
from jax import lax
from jax.experimental import pallas as pl
from jax.experimental.pallas import tpu as pltpu
from jax.experimental.pallas import tpu_sc as plsc

N_NODES = 10000
N_EDGES = 160000
D = 128
BN_EPS = 1e-5
LN_EPS = 1e-5
_BN_SCALE = np.float32(1.0 / np.sqrt(1.0 + BN_EPS))
_INV_SQRT2 = np.float32(1.0 / np.sqrt(2.0))

_NC = 2
_NS = 16
_NW = _NC * _NS
E_PAD = 163840
E_H = E_PAD // 2
_CPW_SS = 20
_NCH_SS = _NW * _CPW_SS
_CPW_G1 = 20
_NCH_G1 = _NW * _CPW_G1
_CPW_G2 = 40
_NCH_G2 = _NW * _CPW_G2
_ACC = N_NODES + 16


def _gelu(x):
    return 0.5 * x * (1.0 + lax.erf(x * _INV_SQRT2))


def _bn(x, g, b):
    return (x * _BN_SCALE) * g + b


def _ln(x, g, b):
    mu = jnp.mean(x, axis=-1, keepdims=True)
    d = x - mu
    var = jnp.mean(d * d, axis=-1, keepdims=True)
    return d * lax.rsqrt(var + LN_EPS) * g + b


def _mm(a, b):
    return jax.lax.dot_general(a, b, (((1,), (0,)), ((), ())),
                               preferred_element_type=jnp.float32)


def _node_mlp(x, p, g1, be1, g2, be2, wpa, wpb):
    def body(x_ref, w1, b1, g1r, be1r, w2, b2, g2r, be2r, wpa_r, wpb_r,
             o_ref, ta_ref, tb_ref):
        h = _gelu(_bn(_mm(x_ref[...], w1[...]) + b1[...], g1r[...], be1r[...]))
        h = _gelu(_bn(_mm(h, w2[...]) + b2[...], g2r[...], be2r[...]))
        o_ref[...] = h
        ta_ref[...] = _mm(h, wpa_r[...])
        tb_ref[...] = _mm(h, wpb_r[...])

    out = jax.ShapeDtypeStruct((N_NODES, D), jnp.float32)
    return pl.pallas_call(
        body,
        out_shape=(out, out, out),
    )(x, p["l1"]["W"], p["l1"]["b"].reshape(1, D), g1, be1,
      p["l2"]["W"], p["l2"]["b"].reshape(1, D), g2, be2, wpa, wpb)


def _edge_mlp(edge_attr, pn, pcat, emb):
    E = edge_attr.shape[0]
    E_T = 4096
    grid = (E // E_T,)

    def body(a_ref, w1, b1, g1, be1, w2, b2, g2, be2,
             embcat_ref, g0c, be0c, wc, bc, g1c, be1c, o_ref):
        a = a_ref[...]
        numer = a[:, : 32 - 4]
        h = _gelu(_bn(_mm(numer, w1[...]) + b1[...], g1[...], be1[...]))
        en = _gelu(_bn(_mm(h, w2[...]) + b2[...], g2[...], be2[...]))
        cat = a[:, 32 - 4:].astype(jnp.int32)
        ohs = []
        for i in range(4):
            ci = cat[:, i][:, None]
            oh = (ci == lax.broadcasted_iota(jnp.int32, (E_T, 8), 1))
            ohs.append(oh.astype(jnp.float32))
        oh_all = jnp.concatenate(ohs, axis=-1)
        ec = _mm(oh_all, embcat_ref[...])
        ec = _gelu(_bn(ec, g0c[...], be0c[...]))
        ec = _gelu(_bn(_mm(ec, wc[...]) + bc[...], g1c[...], be1c[...]))
        o_ref[...] = jnp.concatenate([en, ec], axis=-1)

    embcat = jnp.zeros((32, 32), jnp.float32)
    for i in range(4):
        embcat = lax.dynamic_update_slice(embcat, emb[i], (8 * i, 8 * i))

    espec = pl.BlockSpec((E_T, 32), lambda i: (i, 0))
    ospec = pl.BlockSpec((E_T, D), lambda i: (i, 0))
    full = lambda *s: pl.BlockSpec(s, lambda i: tuple(0 for _ in s))
    return pl.pallas_call(
        body,
        grid=grid,
        in_specs=[espec,
                  full(28, 96), full(1, 96), full(1, 96), full(1, 96),
                  full(96, 96), full(1, 96), full(1, 96), full(1, 96),
                  full(32, 32), full(1, 32), full(1, 32),
                  full(32, 32), full(1, 32), full(1, 32), full(1, 32)],
        out_specs=ospec,
        out_shape=jax.ShapeDtypeStruct((E, D), jnp.float32),
    )(edge_attr,
      pn["l1"]["W"], pn["l1"]["b"].reshape(1, 96), pn["n1"]["g"].reshape(1, 96), pn["n1"]["be"].reshape(1, 96),
      pn["l2"]["W"], pn["l2"]["b"].reshape(1, 96), pn["n2"]["g"].reshape(1, 96), pn["n2"]["be"].reshape(1, 96),
      embcat, pcat["n0"]["g"].reshape(1, 32), pcat["n0"]["be"].reshape(1, 32),
      pcat["l"]["W"], pcat["l"]["b"].reshape(1, 32),
      pcat["n1"]["g"].reshape(1, 32), pcat["n1"]["be"].reshape(1, 32))


def _edge_stage(edge_emb, s, scol, w, b, g, be, residual):
    E = edge_emb.shape[0]
    E_T = 4096
    grid = (E // E_T,)

    def body(e_ref, s_ref, w_ref, b_ref, g_ref, be_ref, o_ref):
        e = e_ref[...]
        y = _gelu(_ln(_mm(e, w_ref[...]) + s_ref[...] + b_ref[...],
                      g_ref[...], be_ref[...]))
        o_ref[...] = e + y if residual else y

    espec = pl.BlockSpec((E_T, D), lambda i: (i, 0))
    sspec = pl.BlockSpec((E_T, D), lambda i: (i, scol))
    full = lambda *sh: pl.BlockSpec(sh, lambda i: tuple(0 for _ in sh))
    return pl.pallas_call(
        body,
        grid=grid,
        in_specs=[espec, sspec, full(D, D), full(1, D), full(1, D), full(1, D)],
        out_specs=espec,
        out_shape=jax.ShapeDtypeStruct((E, D), jnp.float32),
    )(edge_emb, s, w, b.reshape(1, D), g.reshape(1, D), be.reshape(1, D))


def _node_update(node_emb, aggA, aggB, wn1, wn2, b, g, be, wpa, wpb):
    N_T = 2000
    grid = (N_NODES // N_T,)

    def body(n_ref, a_ref, b2_ref, w1, w2, b_ref, g_ref, be_ref,
             wpa_r, wpb_r, o_ref, ta_ref, tb_ref):
        n = n_ref[...]
        agg = a_ref[0] + a_ref[1] + b2_ref[0] + b2_ref[1]
        u = _gelu(_ln(_mm(n, w1[...]) + _mm(agg, w2[...]) + b_ref[...],
                      g_ref[...], be_ref[...]))
        nn = n + u
        o_ref[...] = nn
        ta_ref[...] = _mm(nn, wpa_r[...])
        tb_ref[...] = _mm(nn, wpb_r[...])

    nspec = pl.BlockSpec((N_T, D), lambda i: (i, 0))
    aspec = pl.BlockSpec((2, N_T, D), lambda i: (0, i, 0))
    t2spec = pl.BlockSpec((N_T, 2 * D), lambda i: (i, 0))
    full = lambda *sh: pl.BlockSpec(sh, lambda i: tuple(0 for _ in sh))
    return pl.pallas_call(
        body,
        grid=grid,
        in_specs=[nspec, aspec, aspec, full(D, D), full(D, D), full(1, D),
                  full(1, D), full(1, D), full(D, 2 * D), full(D, 2 * D)],
        out_specs=(nspec, t2spec, t2spec),
        out_shape=(jax.ShapeDtypeStruct((N_NODES, D), jnp.float32),
                   jax.ShapeDtypeStruct((N_NODES, 2 * D), jnp.float32),
                   jax.ShapeDtypeStruct((N_NODES, 2 * D), jnp.float32)),
    )(node_emb, aggA, aggB, wn1, wn2, b.reshape(1, D), g.reshape(1, D),
      be.reshape(1, D), wpa, wpb)


def _node_update_f(node_emb, aggA, aggB, wn1, wn2, b, g, be, node_pre,
                   wpa1, wpa2, wpb1, wpb2):
    N_T = 2000
    grid = (N_NODES // N_T,)

    def body(n_ref, a_ref, b2_ref, w1, w2, b_ref, g_ref, be_ref, p_ref,
             wpa1_r, wpa2_r, wpb1_r, wpb2_r, ta_ref, tb_ref):
        n = n_ref[...]
        agg = a_ref[0] + a_ref[1] + b2_ref[0] + b2_ref[1]
        u = _gelu(_ln(_mm(n, w1[...]) + _mm(agg, w2[...]) + b_ref[...],
                      g_ref[...], be_ref[...]))
        nn = n + u
        nf = nn + p_ref[...]
        ta_ref[...] = jnp.concatenate(
            [_mm(nn, wpa1_r[...]), _mm(nf, wpa2_r[...])], axis=-1)
        tb_ref[...] = jnp.concatenate(
            [_mm(nn, wpb1_r[...]), _mm(nf, wpb2_r[...])], axis=-1)

    nspec = pl.BlockSpec((N_T, D), lambda i: (i, 0))
    aspec = pl.BlockSpec((2, N_T, D), lambda i: (0, i, 0))
    t2spec = pl.BlockSpec((N_T, 2 * D), lambda i: (i, 0))
    full = lambda *sh: pl.BlockSpec(sh, lambda i: tuple(0 for _ in sh))
    return pl.pallas_call(
        body,
        grid=grid,
        in_specs=[nspec, aspec, aspec, full(D, D), full(D, D), full(1, D),
                  full(1, D), full(1, D), nspec,
                  full(D, D), full(D, D), full(D, D), full(D, D)],
        out_specs=(t2spec, t2spec),
        out_shape=(jax.ShapeDtypeStruct((N_NODES, 2 * D), jnp.float32),
                   jax.ShapeDtypeStruct((N_NODES, 2 * D), jnp.float32)),
    )(node_emb, aggA, aggB, wn1, wn2, b.reshape(1, D), g.reshape(1, D),
      be.reshape(1, D), node_pre, wpa1, wpa2, wpb1, wpb2)


def _final_stage(edge_emb, edge_pre, s, scol, w_e, b1, g1, be1, w2, b2):
    E = edge_emb.shape[0]
    E_T = 4096
    grid = (E // E_T,)

    def body(e_ref, ep_ref, s_ref, w_ref, b1_ref, g_ref, be_ref,
             w2_ref, b2_ref, o_ref):
        e = e_ref[...] + ep_ref[...]
        h = _gelu(_bn(_mm(e, w_ref[...]) + s_ref[...] + b1_ref[...],
                      g_ref[...], be_ref[...]))
        o_ref[...] = _mm(h, w2_ref[...]) + b2_ref[...]

    espec = pl.BlockSpec((E_T, D), lambda i: (i, 0))
    sspec = pl.BlockSpec((E_T, D), lambda i: (i, scol))
    full = lambda *sh: pl.BlockSpec(sh, lambda i: tuple(0 for _ in sh))
    return pl.pallas_call(
        body,
        grid=grid,
        in_specs=[espec, espec, sspec, full(D, D), full(1, D), full(1, D),
                  full(1, D), full(D, 3), full(1, 3)],
        out_specs=pl.BlockSpec((E_T, 3), lambda i: (i, 0)),
        out_shape=jax.ShapeDtypeStruct((E, 3), jnp.float32),
    )(edge_emb, edge_pre, s, w_e, b1.reshape(1, D), g1.reshape(1, D),
      be1.reshape(1, D), w2, b2.reshape(1, 3))


@functools.cache
def _sc_kernels():
    mesh = plsc.VectorSubcoreMesh(
        core_axis_name="c", subcore_axis_name="s",
        num_cores=_NC, num_subcores=_NS)

    def make_gather(cpw, ch, width):
        @functools.partial(
            pl.kernel,
            out_type=jax.ShapeDtypeStruct((_NW * cpw * ch, width),
                                          jnp.float32),
            mesh=mesh,
            scratch_types=[
                pltpu.VMEM((cpw, ch), jnp.int32),
                pltpu.VMEM((cpw, ch), jnp.int32),
                pltpu.VMEM((ch, width), jnp.float32),
                pltpu.VMEM((ch, width), jnp.float32),
                pltpu.VMEM((ch, width), jnp.float32),
                pltpu.VMEM((ch, width), jnp.float32),
                pltpu.VMEM((ch, width), jnp.float32),
                pltpu.VMEM((ch, width), jnp.float32),
            ] + [pltpu.SemaphoreType.DMA] * 9,
        )
        def gather2add(ta_hbm, tb_hbm, ia_hbm, ib_hbm, out_hbm,
                       ia_v, ib_v, ba0, bb0, ba1, bb1, ba2, bb2,
                       sa0, sb0, sa1, sb1, sa2, sb2, so0, so1, so2):
            c = lax.axis_index("c")
            s = lax.axis_index("s")
            w = s * _NC + c
            first = w * cpw
            pltpu.sync_copy(ia_hbm.at[w], ia_v)
            pltpu.sync_copy(ib_hbm.at[w], ib_v)

            ba = [ba0, ba1, ba2]
            bb = [bb0, bb1, bb2]
            sa = [sa0, sa1, sa2]
            sb = [sb0, sb1, sb2]
            so = [so0, so1, so2]

            def start(k):
                t = k % 3
                pltpu.async_copy(ta_hbm.at[ia_v.at[k]], ba[t], sa[t])
                pltpu.async_copy(tb_hbm.at[ib_v.at[k]], bb[t], sb[t])

            def finish(k):
                t = k % 3
                pltpu.make_async_copy(ta_hbm.at[ia_v.at[k]], ba[t],
                                      sa[t]).wait()
                pltpu.make_async_copy(tb_hbm.at[ib_v.at[k]], bb[t],
                                      sb[t]).wait()
                bav, bbv = ba[t], bb[t]

                def addrow(r, carry):
                    for cc in range(width // 16):
                        sl = pl.ds(cc * 16, 16)
                        plsc.addupdate(bav.at[r, sl], bbv[r, sl])
                    return carry

                lax.fori_loop(0, ch, addrow, 0)
                pltpu.async_copy(bav, out_hbm.at[pl.ds((first + k) * ch, ch)],
                                 so[t])

            def drain_out(k):
                t = k % 3
                pltpu.make_async_copy(ba[t], out_hbm.at[pl.ds(0, ch)],
                                      so[t]).wait()

            for k in range(cpw + 2):
                if k < cpw:
                    if k >= 3:
                        drain_out(k - 3)
                    start(k)
                if k >= 2:
                    finish(k - 2)
            for k in range(cpw - 3, cpw):
                drain_out(k)

        return gather2add

    @functools.partial(
        pl.kernel,
        out_type=jax.ShapeDtypeStruct((_NC, N_NODES, D), jnp.float32),
        mesh=mesh,
        scratch_types=[
            pltpu.VMEM((_CPW_SS, 128), jnp.int32),
            pltpu.VMEM((128, D), jnp.float32),
            pltpu.VMEM((128, D), jnp.float32),
            pltpu.VMEM_SHARED((_ACC, D), jnp.float32),
        ] + [pltpu.SemaphoreType.DMA] * 4,
    )
    def segsum(m_hbm, dst_hbm, out_hbm, idx_v, bm0, bm1, acc_sh,
               sm0, sm1, ss0, ss1):
        c = lax.axis_index("c")
        s = lax.axis_index("s")
        w = s * _NC + c
        first = w * _CPW_SS

        def zrow(r, carry):
            for cc in range(D // 16):
                bm0[r, pl.ds(cc * 16, 16)] = jnp.zeros((16,), jnp.float32)
            return carry

        lax.fori_loop(0, 128, zrow, 0)
        row0 = s * 624
        for k in range(4):
            pltpu.sync_copy(bm0, acc_sh.at[pl.ds(row0 + k * 128, 128)])
        pltpu.sync_copy(bm0.at[pl.ds(0, 112)],
                        acc_sh.at[pl.ds(row0 + 512, 112)])

        @pl.when(s == _NS - 1)
        def _zero_tail():
            pltpu.sync_copy(bm0.at[pl.ds(0, 32)],
                            acc_sh.at[pl.ds(_ACC - 32, 32)])

        pltpu.sync_copy(dst_hbm.at[w], idx_v)
        plsc.subcore_barrier()

        bm = [bm0, bm1]
        sm = [sm0, sm1]
        ss = [ss0, ss1]

        def load(k):
            t = k % 2
            pltpu.async_copy(m_hbm.at[pl.ds((first + k) * 128, 128)],
                             bm[t], sm[t])

        def scat(k):
            t = k % 2
            pltpu.make_async_copy(m_hbm.at[pl.ds(0, 128)], bm[t], sm[t]).wait()
            pltpu.async_copy(bm[t], acc_sh.at[idx_v.at[k]], ss[t], add=True)

        def drain_scat(k):
            t = k % 2
            pltpu.make_async_copy(bm[t], acc_sh.at[pl.ds(0, 128)],
                                  ss[t]).wait()

        for k in range(_CPW_SS + 1):
            if k < _CPW_SS:
                if k >= 2:
                    drain_scat(k - 2)
                load(k)
            if k >= 1:
                scat(k - 1)
        for k in range(_CPW_SS - 2, _CPW_SS):
            drain_scat(k)
        plsc.subcore_barrier()

        def out_rows(r, n):
            pltpu.sync_copy(acc_sh.at[pl.ds(r, n)], bm0.at[pl.ds(0, n)])
            pltpu.sync_copy(bm0.at[pl.ds(0, n)], out_hbm.at[c, pl.ds(r, n)])

        for k in range(4):
            out_rows(row0 + k * 128, 128)
        out_rows(row0 + 512, 112)

        @pl.when(s == _NS - 1)
        def _out_tail():
            out_rows(N_NODES - 16, 16)

    return make_gather(_CPW_G1, 128, D), make_gather(_CPW_G2, 64, 2 * D), segsum


def _sc_gather2add(ta, tb, ia, ib):
    return _sc_kernels()[0](ta, tb, ia, ib)


def _sc_gather2add256(ta, tb, ia, ib):
    return _sc_kernels()[1](ta, tb, ia, ib)


def _sc_segsum(m, dst):
    return _sc_kernels()[2](m, dst)


def kernel(x, edge_attr, params, edge_index):
    pad_n = E_PAD - N_EDGES
    spread = (jnp.arange(pad_n, dtype=jnp.int32) * 37) % N_NODES
    src_f = jnp.concatenate([edge_index[0], spread])
    dst_f = jnp.concatenate([edge_index[1], spread])
    dsts_f = jnp.concatenate(
        [edge_index[1],
         N_NODES + (jnp.arange(pad_n, dtype=jnp.int32) % 16)])
    halves = []
    for h in range(2):
        sl = slice(h * E_H, (h + 1) * E_H)
        halves.append(dict(
            src=src_f[sl].reshape(_NW, _CPW_G1, 128),
            dst=dst_f[sl].reshape(_NW, _CPW_G1, 128),
            src64=src_f[sl].reshape(_NW, _CPW_G2, 64),
            dst64=dst_f[sl].reshape(_NW, _CPW_G2, 64),
            dsts=dsts_f[sl].reshape(_NW, _CPW_SS, 128),
        ))
    edge_attr = jnp.pad(edge_attr, ((0, pad_n), (0, 0)))

    gnn = params["gnn"]
    msgW = [lp["msg"]["W"] for lp in gnn]
    edgW = [lp["edge"]["W"] for lp in gnn]
    nodW = [lp["node"]["W"] for lp in gnn]
    finW = params["final"]["l1"]["W"]

    pm = params["node_mlp"]
    node_emb, ta, tb = _node_mlp(
        x, pm, pm["n1"]["g"].reshape(1, D), pm["n1"]["be"].reshape(1, D),
        pm["n2"]["g"].reshape(1, D), pm["n2"]["be"].reshape(1, D),
        msgW[0][0:D], msgW[0][D:2 * D])
    node_pre = node_emb

    ee = [_edge_mlp(edge_attr[h * E_H:(h + 1) * E_H], params["edge_numer"],
                    params["edge_cat_mlp"], params["cat_emb"])
          for h in range(2)]
    edge_pre = list(ee)

    s1 = [_sc_gather2add(ta, tb, halves[h]["dst"], halves[h]["src"])
          for h in range(2)]
    s1col = 0
    for l in range(3):
        lp = gnn[l]
        m0 = _edge_stage(ee[0], s1[0], s1col, msgW[l][2 * D:3 * D],
                         lp["msg"]["b"], lp["msg"]["g"], lp["msg"]["be"],
                         residual=False)
        aggA = _sc_segsum(m0, halves[0]["dsts"])
        m1 = _edge_stage(ee[1], s1[1], s1col, msgW[l][2 * D:3 * D],
                         lp["msg"]["b"], lp["msg"]["g"], lp["msg"]["be"],
                         residual=False)
        aggB = _sc_segsum(m1, halves[1]["dsts"])
        if l < 2:
            wpa = jnp.concatenate([edgW[l][D:2 * D], msgW[l + 1][0:D]], axis=1)
            wpb = jnp.concatenate([edgW[l][2 * D:3 * D], msgW[l + 1][D:2 * D]],
                                  axis=1)
            node_emb, Ta, Tb = _node_update(
                node_emb, aggA, aggB, nodW[l][0:D], nodW[l][D:2 * D],
                lp["node"]["b"], lp["node"]["g"], lp["node"]["be"], wpa, wpb)
        else:
            Ta, Tb = _node_update_f(
                node_emb, aggA, aggB, nodW[l][0:D], nodW[l][D:2 * D],
                lp["node"]["b"], lp["node"]["g"], lp["node"]["be"], node_pre,
                edgW[l][D:2 * D], finW[D:2 * D],
                edgW[l][2 * D:3 * D], finW[0:D])
        g0 = _sc_gather2add256(Ta, Tb, halves[0]["dst64"], halves[0]["src64"])
        ee[0] = _edge_stage(ee[0], g0, 0, edgW[l][0:D], lp["edge"]["b"],
                            lp["edge"]["g"], lp["edge"]["be"], residual=True)
        g1 = _sc_gather2add256(Ta, Tb, halves[1]["dst64"], halves[1]["src64"])
        ee[1] = _edge_stage(ee[1], g1, 0, edgW[l][0:D], lp["edge"]["b"],
                            lp["edge"]["g"], lp["edge"]["be"], residual=True)
        s1, s1col = [g0, g1], 1

    pf = params["final"]
    out = [_final_stage(ee[h], edge_pre[h], s1[h], 1, finW[2 * D:3 * D],
                        pf["l1"]["b"], pf["n1"]["g"], pf["n1"]["be"],
                        pf["l2"]["W"], pf["l2"]["b"])
           for h in range(2)]
    return jnp.concatenate(out)[:N_EDGES]


# speedup vs baseline: 1.2020x; 1.0131x over previous
import functools

import jax
import jax.numpy as jnp
import numpy as np
from jax import lax
from jax.experimental import pallas as pl
from jax.experimental.pallas import tpu as pltpu
from jax.experimental.pallas import tpu_sc as plsc

N_NODES = 10000
N_EDGES = 160000
D = 128
BN_EPS = 1e-5
LN_EPS = 1e-5
_BN_SCALE = np.float32(1.0 / np.sqrt(1.0 + BN_EPS))
_INV_SQRT2 = np.float32(1.0 / np.sqrt(2.0))

_NC = 2
_NS = 16
_NW = _NC * _NS
E_PAD = 163840
E_H = E_PAD // 2
_CPW_SS = 20
_NCH_SS = _NW * _CPW_SS
_CPW_G1 = 20
_NCH_G1 = _NW * _CPW_G1
_CPW_G2 = 40
_NCH_G2 = _NW * _CPW_G2
_ACC = N_NODES + 16


def _gelu(x):
    return 0.5 * x * (1.0 + lax.erf(x * _INV_SQRT2))


def _bn(x, g, b):
    return (x * _BN_SCALE) * g + b


def _ln(x, g, b):
    mu = jnp.mean(x, axis=-1, keepdims=True)
    d = x - mu
    var = jnp.mean(d * d, axis=-1, keepdims=True)
    return d * lax.rsqrt(var + LN_EPS) * g + b


def _mm(a, b):
    return jax.lax.dot_general(a, b, (((1,), (0,)), ((), ())),
                               preferred_element_type=jnp.float32)


def _node_mlp(x, p, g1, be1, g2, be2, wpa, wpb):
    def body(x_ref, w1, b1, g1r, be1r, w2, b2, g2r, be2r, wpa_r, wpb_r,
             o_ref, ta_ref, tb_ref):
        h = _gelu(_bn(_mm(x_ref[...], w1[...]) + b1[...], g1r[...], be1r[...]))
        h = _gelu(_bn(_mm(h, w2[...]) + b2[...], g2r[...], be2r[...]))
        o_ref[...] = h
        ta_ref[...] = _mm(h, wpa_r[...])
        tb_ref[...] = _mm(h, wpb_r[...])

    out = jax.ShapeDtypeStruct((N_NODES, D), jnp.float32)
    return pl.pallas_call(
        body,
        out_shape=(out, out, out),
    )(x, p["l1"]["W"], p["l1"]["b"].reshape(1, D), g1, be1,
      p["l2"]["W"], p["l2"]["b"].reshape(1, D), g2, be2, wpa, wpb)


def _edge_mlp(edge_attr, pn, pcat, emb):
    E = edge_attr.shape[0]
    E_T = 8192
    grid = (E // E_T,)

    def body(a_ref, w1, b1, g1, be1, w2, b2, g2, be2,
             embcat_ref, g0c, be0c, wc, bc, g1c, be1c, o_ref):
        a = a_ref[...]
        numer = a[:, : 32 - 4]
        h = _gelu(_bn(_mm(numer, w1[...]) + b1[...], g1[...], be1[...]))
        en = _gelu(_bn(_mm(h, w2[...]) + b2[...], g2[...], be2[...]))
        cat = a[:, 32 - 4:].astype(jnp.int32)
        ohs = []
        for i in range(4):
            ci = cat[:, i][:, None]
            oh = (ci == lax.broadcasted_iota(jnp.int32, (E_T, 8), 1))
            ohs.append(oh.astype(jnp.float32))
        oh_all = jnp.concatenate(ohs, axis=-1)
        ec = _mm(oh_all, embcat_ref[...])
        ec = _gelu(_bn(ec, g0c[...], be0c[...]))
        ec = _gelu(_bn(_mm(ec, wc[...]) + bc[...], g1c[...], be1c[...]))
        o_ref[...] = jnp.concatenate([en, ec], axis=-1)

    embcat = jnp.zeros((32, 32), jnp.float32)
    for i in range(4):
        embcat = lax.dynamic_update_slice(embcat, emb[i], (8 * i, 8 * i))

    espec = pl.BlockSpec((E_T, 32), lambda i: (i, 0))
    ospec = pl.BlockSpec((E_T, D), lambda i: (i, 0))
    full = lambda *s: pl.BlockSpec(s, lambda i: tuple(0 for _ in s))
    return pl.pallas_call(
        body,
        grid=grid,
        in_specs=[espec,
                  full(28, 96), full(1, 96), full(1, 96), full(1, 96),
                  full(96, 96), full(1, 96), full(1, 96), full(1, 96),
                  full(32, 32), full(1, 32), full(1, 32),
                  full(32, 32), full(1, 32), full(1, 32), full(1, 32)],
        out_specs=ospec,
        out_shape=jax.ShapeDtypeStruct((E, D), jnp.float32),
    )(edge_attr,
      pn["l1"]["W"], pn["l1"]["b"].reshape(1, 96), pn["n1"]["g"].reshape(1, 96), pn["n1"]["be"].reshape(1, 96),
      pn["l2"]["W"], pn["l2"]["b"].reshape(1, 96), pn["n2"]["g"].reshape(1, 96), pn["n2"]["be"].reshape(1, 96),
      embcat, pcat["n0"]["g"].reshape(1, 32), pcat["n0"]["be"].reshape(1, 32),
      pcat["l"]["W"], pcat["l"]["b"].reshape(1, 32),
      pcat["n1"]["g"].reshape(1, 32), pcat["n1"]["be"].reshape(1, 32))


def _edge_stage(edge_emb, s, scol, w, b, g, be, residual):
    E = edge_emb.shape[0]
    E_T = 8192
    grid = (E // E_T,)

    def body(e_ref, s_ref, w_ref, b_ref, g_ref, be_ref, o_ref):
        e = e_ref[...]
        y = _gelu(_ln(_mm(e, w_ref[...]) + s_ref[...] + b_ref[...],
                      g_ref[...], be_ref[...]))
        o_ref[...] = e + y if residual else y

    espec = pl.BlockSpec((E_T, D), lambda i: (i, 0))
    sspec = pl.BlockSpec((E_T, D), lambda i: (i, scol))
    full = lambda *sh: pl.BlockSpec(sh, lambda i: tuple(0 for _ in sh))
    return pl.pallas_call(
        body,
        grid=grid,
        in_specs=[espec, sspec, full(D, D), full(1, D), full(1, D), full(1, D)],
        out_specs=espec,
        out_shape=jax.ShapeDtypeStruct((E, D), jnp.float32),
    )(edge_emb, s, w, b.reshape(1, D), g.reshape(1, D), be.reshape(1, D))


def _node_update(node_emb, aggA, aggB, wn1, wn2, b, g, be, wpa, wpb):
    N_T = 2000
    grid = (N_NODES // N_T,)

    def body(n_ref, a_ref, b2_ref, w1, w2, b_ref, g_ref, be_ref,
             wpa_r, wpb_r, o_ref, ta_ref, tb_ref):
        n = n_ref[...]
        agg = a_ref[0] + a_ref[1] + b2_ref[0] + b2_ref[1]
        u = _gelu(_ln(_mm(n, w1[...]) + _mm(agg, w2[...]) + b_ref[...],
                      g_ref[...], be_ref[...]))
        nn = n + u
        o_ref[...] = nn
        ta_ref[...] = _mm(nn, wpa_r[...])
        tb_ref[...] = _mm(nn, wpb_r[...])

    nspec = pl.BlockSpec((N_T, D), lambda i: (i, 0))
    aspec = pl.BlockSpec((2, N_T, D), lambda i: (0, i, 0))
    t2spec = pl.BlockSpec((N_T, 2 * D), lambda i: (i, 0))
    full = lambda *sh: pl.BlockSpec(sh, lambda i: tuple(0 for _ in sh))
    return pl.pallas_call(
        body,
        grid=grid,
        in_specs=[nspec, aspec, aspec, full(D, D), full(D, D), full(1, D),
                  full(1, D), full(1, D), full(D, 2 * D), full(D, 2 * D)],
        out_specs=(nspec, t2spec, t2spec),
        out_shape=(jax.ShapeDtypeStruct((N_NODES, D), jnp.float32),
                   jax.ShapeDtypeStruct((N_NODES, 2 * D), jnp.float32),
                   jax.ShapeDtypeStruct((N_NODES, 2 * D), jnp.float32)),
    )(node_emb, aggA, aggB, wn1, wn2, b.reshape(1, D), g.reshape(1, D),
      be.reshape(1, D), wpa, wpb)


def _node_update_f(node_emb, aggA, aggB, wn1, wn2, b, g, be, node_pre,
                   wpa1, wpa2, wpb1, wpb2):
    N_T = 2000
    grid = (N_NODES // N_T,)

    def body(n_ref, a_ref, b2_ref, w1, w2, b_ref, g_ref, be_ref, p_ref,
             wpa1_r, wpa2_r, wpb1_r, wpb2_r, ta_ref, tb_ref):
        n = n_ref[...]
        agg = a_ref[0] + a_ref[1] + b2_ref[0] + b2_ref[1]
        u = _gelu(_ln(_mm(n, w1[...]) + _mm(agg, w2[...]) + b_ref[...],
                      g_ref[...], be_ref[...]))
        nn = n + u
        nf = nn + p_ref[...]
        ta_ref[...] = jnp.concatenate(
            [_mm(nn, wpa1_r[...]), _mm(nf, wpa2_r[...])], axis=-1)
        tb_ref[...] = jnp.concatenate(
            [_mm(nn, wpb1_r[...]), _mm(nf, wpb2_r[...])], axis=-1)

    nspec = pl.BlockSpec((N_T, D), lambda i: (i, 0))
    aspec = pl.BlockSpec((2, N_T, D), lambda i: (0, i, 0))
    t2spec = pl.BlockSpec((N_T, 2 * D), lambda i: (i, 0))
    full = lambda *sh: pl.BlockSpec(sh, lambda i: tuple(0 for _ in sh))
    return pl.pallas_call(
        body,
        grid=grid,
        in_specs=[nspec, aspec, aspec, full(D, D), full(D, D), full(1, D),
                  full(1, D), full(1, D), nspec,
                  full(D, D), full(D, D), full(D, D), full(D, D)],
        out_specs=(t2spec, t2spec),
        out_shape=(jax.ShapeDtypeStruct((N_NODES, 2 * D), jnp.float32),
                   jax.ShapeDtypeStruct((N_NODES, 2 * D), jnp.float32)),
    )(node_emb, aggA, aggB, wn1, wn2, b.reshape(1, D), g.reshape(1, D),
      be.reshape(1, D), node_pre, wpa1, wpa2, wpb1, wpb2)


def _final_stage(edge_emb, edge_pre, s, scol, w_e, b1, g1, be1, w2, b2):
    E = edge_emb.shape[0]
    E_T = 8192
    grid = (E // E_T,)

    def body(e_ref, ep_ref, s_ref, w_ref, b1_ref, g_ref, be_ref,
             w2_ref, b2_ref, o_ref):
        e = e_ref[...] + ep_ref[...]
        h = _gelu(_bn(_mm(e, w_ref[...]) + s_ref[...] + b1_ref[...],
                      g_ref[...], be_ref[...]))
        o_ref[...] = _mm(h, w2_ref[...]) + b2_ref[...]

    espec = pl.BlockSpec((E_T, D), lambda i: (i, 0))
    sspec = pl.BlockSpec((E_T, D), lambda i: (i, scol))
    full = lambda *sh: pl.BlockSpec(sh, lambda i: tuple(0 for _ in sh))
    return pl.pallas_call(
        body,
        grid=grid,
        in_specs=[espec, espec, sspec, full(D, D), full(1, D), full(1, D),
                  full(1, D), full(D, 3), full(1, 3)],
        out_specs=pl.BlockSpec((E_T, 3), lambda i: (i, 0)),
        out_shape=jax.ShapeDtypeStruct((E, 3), jnp.float32),
    )(edge_emb, edge_pre, s, w_e, b1.reshape(1, D), g1.reshape(1, D),
      be1.reshape(1, D), w2, b2.reshape(1, 3))


@functools.cache
def _sc_kernels():
    mesh = plsc.VectorSubcoreMesh(
        core_axis_name="c", subcore_axis_name="s",
        num_cores=_NC, num_subcores=_NS)

    def make_gather(cpw, ch, width):
        @functools.partial(
            pl.kernel,
            out_type=jax.ShapeDtypeStruct((_NW * cpw * ch, width),
                                          jnp.float32),
            mesh=mesh,
            scratch_types=[
                pltpu.VMEM((cpw, ch), jnp.int32),
                pltpu.VMEM((cpw, ch), jnp.int32),
                pltpu.VMEM((ch, width), jnp.float32),
                pltpu.VMEM((ch, width), jnp.float32),
                pltpu.VMEM((ch, width), jnp.float32),
                pltpu.VMEM((ch, width), jnp.float32),
                pltpu.VMEM((ch, width), jnp.float32),
                pltpu.VMEM((ch, width), jnp.float32),
            ] + [pltpu.SemaphoreType.DMA] * 9,
        )
        def gather2add(ta_hbm, tb_hbm, ia_hbm, ib_hbm, out_hbm,
                       ia_v, ib_v, ba0, bb0, ba1, bb1, ba2, bb2,
                       sa0, sb0, sa1, sb1, sa2, sb2, so0, so1, so2):
            c = lax.axis_index("c")
            s = lax.axis_index("s")
            w = s * _NC + c
            first = w * cpw
            pltpu.sync_copy(ia_hbm.at[w], ia_v)
            pltpu.sync_copy(ib_hbm.at[w], ib_v)

            ba = [ba0, ba1, ba2]
            bb = [bb0, bb1, bb2]
            sa = [sa0, sa1, sa2]
            sb = [sb0, sb1, sb2]
            so = [so0, so1, so2]

            def start(k):
                t = k % 3
                pltpu.async_copy(ta_hbm.at[ia_v.at[k]], ba[t], sa[t])
                pltpu.async_copy(tb_hbm.at[ib_v.at[k]], bb[t], sb[t])

            def finish(k):
                t = k % 3
                pltpu.make_async_copy(ta_hbm.at[ia_v.at[k]], ba[t],
                                      sa[t]).wait()
                pltpu.make_async_copy(tb_hbm.at[ib_v.at[k]], bb[t],
                                      sb[t]).wait()
                bav, bbv = ba[t], bb[t]

                def addrow(r, carry):
                    for cc in range(width // 16):
                        sl = pl.ds(cc * 16, 16)
                        plsc.addupdate(bav.at[r, sl], bbv[r, sl])
                    return carry

                lax.fori_loop(0, ch, addrow, 0)
                pltpu.async_copy(bav, out_hbm.at[pl.ds((first + k) * ch, ch)],
                                 so[t])

            def drain_out(k):
                t = k % 3
                pltpu.make_async_copy(ba[t], out_hbm.at[pl.ds(0, ch)],
                                      so[t]).wait()

            for k in range(cpw + 2):
                if k < cpw:
                    if k >= 3:
                        drain_out(k - 3)
                    start(k)
                if k >= 2:
                    finish(k - 2)
            for k in range(cpw - 3, cpw):
                drain_out(k)

        return gather2add

    @functools.partial(
        pl.kernel,
        out_type=jax.ShapeDtypeStruct((_NC, N_NODES, D), jnp.float32),
        mesh=mesh,
        scratch_types=[
            pltpu.VMEM((_CPW_SS, 128), jnp.int32),
            pltpu.VMEM((128, D), jnp.float32),
            pltpu.VMEM((128, D), jnp.float32),
            pltpu.VMEM_SHARED((_ACC, D), jnp.float32),
        ] + [pltpu.SemaphoreType.DMA] * 4,
    )
    def segsum(m_hbm, dst_hbm, out_hbm, idx_v, bm0, bm1, acc_sh,
               sm0, sm1, ss0, ss1):
        c = lax.axis_index("c")
        s = lax.axis_index("s")
        w = s * _NC + c
        first = w * _CPW_SS

        def zrow(r, carry):
            for cc in range(D // 16):
                bm0[r, pl.ds(cc * 16, 16)] = jnp.zeros((16,), jnp.float32)
            return carry

        lax.fori_loop(0, 128, zrow, 0)
        row0 = s * 624
        for k in range(4):
            pltpu.sync_copy(bm0, acc_sh.at[pl.ds(row0 + k * 128, 128)])
        pltpu.sync_copy(bm0.at[pl.ds(0, 112)],
                        acc_sh.at[pl.ds(row0 + 512, 112)])

        @pl.when(s == _NS - 1)
        def _zero_tail():
            pltpu.sync_copy(bm0.at[pl.ds(0, 32)],
                            acc_sh.at[pl.ds(_ACC - 32, 32)])

        pltpu.sync_copy(dst_hbm.at[w], idx_v)
        plsc.subcore_barrier()

        bm = [bm0, bm1]
        sm = [sm0, sm1]
        ss = [ss0, ss1]

        def load(k):
            t = k % 2
            pltpu.async_copy(m_hbm.at[pl.ds((first + k) * 128, 128)],
                             bm[t], sm[t])

        def scat(k):
            t = k % 2
            pltpu.make_async_copy(m_hbm.at[pl.ds(0, 128)], bm[t], sm[t]).wait()
            pltpu.async_copy(bm[t], acc_sh.at[idx_v.at[k]], ss[t], add=True)

        def drain_scat(k):
            t = k % 2
            pltpu.make_async_copy(bm[t], acc_sh.at[pl.ds(0, 128)],
                                  ss[t]).wait()

        for k in range(_CPW_SS + 1):
            if k < _CPW_SS:
                if k >= 2:
                    drain_scat(k - 2)
                load(k)
            if k >= 1:
                scat(k - 1)
        for k in range(_CPW_SS - 2, _CPW_SS):
            drain_scat(k)
        plsc.subcore_barrier()

        def out_rows(r, n):
            pltpu.sync_copy(acc_sh.at[pl.ds(r, n)], bm0.at[pl.ds(0, n)])
            pltpu.sync_copy(bm0.at[pl.ds(0, n)], out_hbm.at[c, pl.ds(r, n)])

        for k in range(4):
            out_rows(row0 + k * 128, 128)
        out_rows(row0 + 512, 112)

        @pl.when(s == _NS - 1)
        def _out_tail():
            out_rows(N_NODES - 16, 16)

    return make_gather(_CPW_G1, 128, D), make_gather(_CPW_G2, 64, 2 * D), segsum


def _sc_gather2add(ta, tb, ia, ib):
    return _sc_kernels()[0](ta, tb, ia, ib)


def _sc_gather2add256(ta, tb, ia, ib):
    return _sc_kernels()[1](ta, tb, ia, ib)


def _sc_segsum(m, dst):
    return _sc_kernels()[2](m, dst)


def kernel(x, edge_attr, params, edge_index):
    pad_n = E_PAD - N_EDGES
    spread = (jnp.arange(pad_n, dtype=jnp.int32) * 37) % N_NODES
    src_f = jnp.concatenate([edge_index[0], spread])
    dst_f = jnp.concatenate([edge_index[1], spread])
    dsts_f = jnp.concatenate(
        [edge_index[1],
         N_NODES + (jnp.arange(pad_n, dtype=jnp.int32) % 16)])
    halves = []
    for h in range(2):
        sl = slice(h * E_H, (h + 1) * E_H)
        halves.append(dict(
            src=src_f[sl].reshape(_NW, _CPW_G1, 128),
            dst=dst_f[sl].reshape(_NW, _CPW_G1, 128),
            src64=src_f[sl].reshape(_NW, _CPW_G2, 64),
            dst64=dst_f[sl].reshape(_NW, _CPW_G2, 64),
            dsts=dsts_f[sl].reshape(_NW, _CPW_SS, 128),
        ))
    edge_attr = jnp.pad(edge_attr, ((0, pad_n), (0, 0)))

    gnn = params["gnn"]
    msgW = [lp["msg"]["W"] for lp in gnn]
    edgW = [lp["edge"]["W"] for lp in gnn]
    nodW = [lp["node"]["W"] for lp in gnn]
    finW = params["final"]["l1"]["W"]

    pm = params["node_mlp"]
    node_emb, ta, tb = _node_mlp(
        x, pm, pm["n1"]["g"].reshape(1, D), pm["n1"]["be"].reshape(1, D),
        pm["n2"]["g"].reshape(1, D), pm["n2"]["be"].reshape(1, D),
        msgW[0][0:D], msgW[0][D:2 * D])
    node_pre = node_emb

    ee = [_edge_mlp(edge_attr[h * E_H:(h + 1) * E_H], params["edge_numer"],
                    params["edge_cat_mlp"], params["cat_emb"])
          for h in range(2)]
    edge_pre = list(ee)

    s1 = [_sc_gather2add(ta, tb, halves[h]["dst"], halves[h]["src"])
          for h in range(2)]
    s1col = 0
    for l in range(3):
        lp = gnn[l]
        m0 = _edge_stage(ee[0], s1[0], s1col, msgW[l][2 * D:3 * D],
                         lp["msg"]["b"], lp["msg"]["g"], lp["msg"]["be"],
                         residual=False)
        aggA = _sc_segsum(m0, halves[0]["dsts"])
        m1 = _edge_stage(ee[1], s1[1], s1col, msgW[l][2 * D:3 * D],
                         lp["msg"]["b"], lp["msg"]["g"], lp["msg"]["be"],
                         residual=False)
        aggB = _sc_segsum(m1, halves[1]["dsts"])
        if l < 2:
            wpa = jnp.concatenate([edgW[l][D:2 * D], msgW[l + 1][0:D]], axis=1)
            wpb = jnp.concatenate([edgW[l][2 * D:3 * D], msgW[l + 1][D:2 * D]],
                                  axis=1)
            node_emb, Ta, Tb = _node_update(
                node_emb, aggA, aggB, nodW[l][0:D], nodW[l][D:2 * D],
                lp["node"]["b"], lp["node"]["g"], lp["node"]["be"], wpa, wpb)
        else:
            Ta, Tb = _node_update_f(
                node_emb, aggA, aggB, nodW[l][0:D], nodW[l][D:2 * D],
                lp["node"]["b"], lp["node"]["g"], lp["node"]["be"], node_pre,
                edgW[l][D:2 * D], finW[D:2 * D],
                edgW[l][2 * D:3 * D], finW[0:D])
        g0 = _sc_gather2add256(Ta, Tb, halves[0]["dst64"], halves[0]["src64"])
        ee[0] = _edge_stage(ee[0], g0, 0, edgW[l][0:D], lp["edge"]["b"],
                            lp["edge"]["g"], lp["edge"]["be"], residual=True)
        g1 = _sc_gather2add256(Ta, Tb, halves[1]["dst64"], halves[1]["src64"])
        ee[1] = _edge_stage(ee[1], g1, 0, edgW[l][0:D], lp["edge"]["b"],
                            lp["edge"]["g"], lp["edge"]["be"], residual=True)
        s1, s1col = [g0, g1], 1

    pf = params["final"]
    out = [_final_stage(ee[h], edge_pre[h], s1[h], 1, finW[2 * D:3 * D],
                        pf["l1"]["b"], pf["n1"]["g"], pf["n1"]["be"],
                        pf["l2"]["W"], pf["l2"]["b"])
           for h in range(2)]
    return jnp.concatenate(out)[:N_EDGES]
